# baseline (speedup 1.0000x reference)
import dataclasses
import functools

import jax
import jax.numpy as jnp
from jax import lax
from jax.experimental import pallas as pl
from jax.experimental.pallas import tpu as pltpu
from jax.experimental.pallas import tpu_sc as plsc

B = 4096
S = 200
D = 64
L = 16
NC = 2
NS = 16
NW = NC * NS
RPW = B // NW
SPAD = ((S + L - 1) // L) * L
NG = SPAD // L
C1 = 128
C2 = S - C1
FLAT = RPW * S


def _sc_body(embed_hbm, short_hbm, weight_hbm, bias_hbm, out_hbm,
             idx_v, embed_v, rows0, rows1, rows2, rows3,
             bv0, bv1, bv2, bv3, out_v,
             sem_in, sem_g0, sem_g1, sem_g2, sem_g3):
    wid = lax.axis_index("s") * NC + lax.axis_index("c")
    row0 = wid * RPW
    base = wid * FLAT

    cp_i = pltpu.async_copy(short_hbm.at[pl.ds(base, FLAT)], idx_v, sem_in)
    cp_e = pltpu.async_copy(embed_hbm.at[pl.ds(row0, RPW)], embed_v, sem_in)
    cp_i.wait()
    cp_e.wait()

    def fire(r, rows_v, bv, sem):
        off = r * S
        pltpu.async_copy(weight_hbm.at[idx_v.at[pl.ds(off, C1)]],
                         rows_v.at[pl.ds(0, C1)], sem)
        pltpu.async_copy(weight_hbm.at[idx_v.at[pl.ds(off + C1, C2)]],
                         rows_v.at[pl.ds(C1, C2)], sem)
        pltpu.async_copy(bias_hbm.at[idx_v.at[pl.ds(off, C1)]],
                         bv.at[pl.ds(0, C1)], sem)
        pltpu.async_copy(bias_hbm.at[idx_v.at[pl.ds(off + C1, C2)]],
                         bv.at[pl.ds(C1, C2)], sem)

    def drain(r, rows_v, bv, sem):
        off = r * S
        pltpu.make_async_copy(weight_hbm.at[idx_v.at[pl.ds(off, C1)]],
                              rows_v.at[pl.ds(0, C1)], sem).wait()
        pltpu.make_async_copy(weight_hbm.at[idx_v.at[pl.ds(off + C1, C2)]],
                              rows_v.at[pl.ds(C1, C2)], sem).wait()
        pltpu.make_async_copy(bias_hbm.at[idx_v.at[pl.ds(off, C1)]],
                              bv.at[pl.ds(0, C1)], sem).wait()
        pltpu.make_async_copy(bias_hbm.at[idx_v.at[pl.ds(off + C1, C2)]],
                              bv.at[pl.ds(C1, C2)], sem).wait()

    row_ids = [lax.iota(jnp.int32, L) + g * L for g in range(NG)]

    def compute(r, rows_v, bv):
        r_vec = jnp.full((L,), r, jnp.int32)

        def dbody(d, accs):
            cols = jnp.full((L,), d, jnp.int32)
            e_d = plsc.load_gather(embed_v, [r_vec, cols])
            return tuple(
                accs[g] + plsc.load_gather(rows_v, [row_ids[g], cols]) * e_d
                for g in range(NG))

        accs = lax.fori_loop(
            0, D, dbody, tuple(jnp.zeros((L,), jnp.float32) for _ in range(NG)),
            unroll=8)
        out_off = r * S
        for g in range(NG):
            out_v[pl.ds(out_off + g * L, L)] = accs[g] + bv[pl.ds(g * L, L)]

    bufs = ((rows0, bv0, sem_g0), (rows1, bv1, sem_g1),
            (rows2, bv2, sem_g2), (rows3, bv3, sem_g3))
    for q in range(3):
        fire(q, *bufs[q])

    @pl.loop(0, RPW // 4)
    def _(p):
        for q in range(4):
            r = p * 4 + q
            drain(r, *bufs[q])
            compute(r, *bufs[q][:2])

            @pl.when(r + 3 < RPW)
            def _():
                fire(r + 3, *bufs[(q + 3) % 4])

    pltpu.sync_copy(out_v.at[pl.ds(0, FLAT)], out_hbm.at[pl.ds(base, FLAT)])


_cp = pltpu.CompilerParams()
for _field, _val in (("needs_layout_passes", False),
                     ("use_tc_tiling_on_sc", False)):
    if _field in pltpu.CompilerParams.__dataclass_fields__:
        _cp = dataclasses.replace(_cp, **{_field: _val})


@functools.partial(
    pl.kernel,
    out_type=jax.ShapeDtypeStruct((B * S,), jnp.float32),
    mesh=plsc.VectorSubcoreMesh(core_axis_name="c", subcore_axis_name="s"),
    compiler_params=_cp,
    scratch_types=[
        pltpu.VMEM((FLAT,), jnp.int32),
        pltpu.VMEM((RPW, D), jnp.float32),
        pltpu.VMEM((SPAD, D), jnp.float32),
        pltpu.VMEM((SPAD, D), jnp.float32),
        pltpu.VMEM((SPAD, D), jnp.float32),
        pltpu.VMEM((SPAD, D), jnp.float32),
        pltpu.VMEM((SPAD,), jnp.float32),
        pltpu.VMEM((SPAD,), jnp.float32),
        pltpu.VMEM((SPAD,), jnp.float32),
        pltpu.VMEM((SPAD,), jnp.float32),
        pltpu.VMEM((FLAT + 8,), jnp.float32),
        pltpu.SemaphoreType.DMA,
        pltpu.SemaphoreType.DMA,
        pltpu.SemaphoreType.DMA,
        pltpu.SemaphoreType.DMA,
        pltpu.SemaphoreType.DMA,
    ],
)
def _sc_sparse_linear(embed_hbm, short_hbm, weight_hbm, bias_hbm, out_hbm,
                      idx_v, embed_v, rows0, rows1, rows2, rows3,
                      bv0, bv1, bv2, bv3, out_v,
                      sem_in, sem_g0, sem_g1, sem_g2, sem_g3):
    _sc_body(embed_hbm, short_hbm, weight_hbm, bias_hbm, out_hbm,
             idx_v, embed_v, rows0, rows1, rows2, rows3,
             bv0, bv1, bv2, bv3, out_v,
             sem_in, sem_g0, sem_g1, sem_g2, sem_g3)


@jax.jit
def kernel(embed, shortlist, weight, bias):
    b, s = shortlist.shape
    short_flat = shortlist.astype(jnp.int32).reshape(-1)
    bias_flat = bias.reshape(-1)
    out = _sc_sparse_linear(embed, short_flat, weight, bias_flat)
    return out.reshape(b, s)


# speedup vs baseline: 2.0393x; 2.0393x over previous
import dataclasses
import functools

import jax
import jax.numpy as jnp
from jax import lax
from jax.experimental import pallas as pl
from jax.experimental.pallas import tpu as pltpu
from jax.experimental.pallas import tpu_sc as plsc

B = 4096
S = 200
D = 64
L = 16
NC = 2
NS = 16
NW = NC * NS
RPW = B // NW
SPAD = ((S + L - 1) // L) * L
NG = SPAD // L
C1 = 128
C2 = S - C1
FLAT = RPW * S
W2 = 2 * D
OCH = 16
OCHW = OCH * S


def _sc_body(embed_hbm, gidx_hbm, pidx_hbm, weight_hbm, bias_hbm, out_hbm,
             gidx_v, pidx_v, embed_v, rows0, rows1, bv0, bv1,
             out0, out1, sem_in, sem_g0, sem_g1, sem_out):
    wid = lax.axis_index("s") * NC + lax.axis_index("c")
    row0 = wid * RPW
    base = wid * FLAT

    cp_g = pltpu.async_copy(gidx_hbm.at[pl.ds(base, FLAT)], gidx_v, sem_in)
    cp_p = pltpu.async_copy(pidx_hbm.at[pl.ds(base, FLAT)], pidx_v, sem_in)
    cp_e = pltpu.async_copy(embed_hbm.at[pl.ds(row0, RPW)], embed_v, sem_in)
    cp_g.wait()
    cp_p.wait()
    cp_e.wait()

    def fire(r, rows_v, bv, sem):
        off = r * S
        pltpu.async_copy(weight_hbm.at[gidx_v.at[pl.ds(off, C1)]],
                         rows_v.at[pl.ds(0, C1)], sem)
        pltpu.async_copy(weight_hbm.at[gidx_v.at[pl.ds(off + C1, C2)]],
                         rows_v.at[pl.ds(C1, C2)], sem)
        pltpu.async_copy(bias_hbm.at[pidx_v.at[pl.ds(off, C1)]],
                         bv.at[pl.ds(0, C1)], sem)
        pltpu.async_copy(bias_hbm.at[pidx_v.at[pl.ds(off + C1, C2)]],
                         bv.at[pl.ds(C1, C2)], sem)

    def drain(r, rows_v, bv, sem):
        off = r * S
        pltpu.make_async_copy(weight_hbm.at[gidx_v.at[pl.ds(off, C1)]],
                              rows_v.at[pl.ds(0, C1)], sem).wait()
        pltpu.make_async_copy(weight_hbm.at[gidx_v.at[pl.ds(off + C1, C2)]],
                              rows_v.at[pl.ds(C1, C2)], sem).wait()
        pltpu.make_async_copy(bias_hbm.at[pidx_v.at[pl.ds(off, C1)]],
                              bv.at[pl.ds(0, C1)], sem).wait()
        pltpu.make_async_copy(bias_hbm.at[pidx_v.at[pl.ds(off + C1, C2)]],
                              bv.at[pl.ds(C1, C2)], sem).wait()

    def lane_perm(x, idx):
        return lax.gather(
            x, idx[:, None],
            dimension_numbers=lax.GatherDimensionNumbers(
                offset_dims=(), collapsed_slice_dims=(0,),
                start_index_map=(0,)),
            slice_sizes=(1,),
            mode=lax.GatherScatterMode.PROMISE_IN_BOUNDS)

    bitrev = [int(f"{i:04b}"[::-1], 2) for i in range(L)]

    def compute(r, rows_v, bv, out_v):
        e = [embed_v[r, pl.ds(k * L, L)] for k in range(D // L)]

        @pl.loop(0, NG)
        def _(g):
            s_base = g * L
            hoffs = (pidx_v[pl.ds(r * S + s_base, L)] & 1) * D
            accs = []
            for j in range(L):
                s = s_base + bitrev[j]
                h = hoffs[bitrev[j]]
                w0 = rows_v[s, pl.ds(h, L)]
                acc = w0 * e[0]
                for k in range(1, D // L):
                    acc = acc + rows_v[s, pl.ds(h + k * L, L)] * e[k]
                accs.append(acc)
            lane = lax.iota(jnp.int32, L)
            for h2 in (8, 4, 2, 1):
                mask = (lane & h2) == 0
                xor_idx = lane ^ h2
                nxt = []
                for i in range(0, len(accs), 2):
                    a, b = accs[i], accs[i + 1]
                    nxt.append(jnp.where(mask,
                                         a + lane_perm(a, xor_idx),
                                         b + lane_perm(b, xor_idx)))
                accs = nxt
            loc = (r % OCH) * S + s_base
            out_v[pl.ds(loc, L)] = accs[0] + bv[pl.ds(s_base, L)]

    fire(0, rows0, bv0, sem_g0)

    @pl.loop(0, RPW // 2)
    def _(p):
        r0 = p * 2
        r1 = r0 + 1
        chunk_par = (r0 // OCH) % 2
        fire(r1, rows1, bv1, sem_g1)
        drain(r0, rows0, bv0, sem_g0)

        @pl.when(chunk_par == 0)
        def _():
            compute(r0, rows0, bv0, out0)

        @pl.when(chunk_par == 1)
        def _():
            compute(r0, rows0, bv0, out1)

        @pl.when(p < RPW // 2 - 1)
        def _():
            fire(r0 + 2, rows0, bv0, sem_g0)

        drain(r1, rows1, bv1, sem_g1)

        @pl.when(chunk_par == 0)
        def _():
            compute(r1, rows1, bv1, out0)

        @pl.when(chunk_par == 1)
        def _():
            compute(r1, rows1, bv1, out1)

        @pl.when(r1 % OCH == OCH - 1)
        def _():
            c = r1 // OCH

            @pl.when(c % 2 == 0)
            def _():
                pltpu.async_copy(out0.at[pl.ds(0, OCHW)],
                                 out_hbm.at[pl.ds(base + c * OCHW, OCHW)],
                                 sem_out)

            @pl.when(c % 2 == 1)
            def _():
                pltpu.async_copy(out1.at[pl.ds(0, OCHW)],
                                 out_hbm.at[pl.ds(base + c * OCHW, OCHW)],
                                 sem_out)

            @pl.when(c >= 1)
            def _():
                pltpu.make_async_copy(
                    out0.at[pl.ds(0, OCHW)],
                    out_hbm.at[pl.ds(base + (c - 1) * OCHW, OCHW)],
                    sem_out).wait()

    pltpu.make_async_copy(
        out1.at[pl.ds(0, OCHW)],
        out_hbm.at[pl.ds(base + (RPW // OCH - 1) * OCHW, OCHW)],
        sem_out).wait()


_cp = pltpu.CompilerParams()
for _field, _val in (("needs_layout_passes", False),
                     ("use_tc_tiling_on_sc", True)):
    if _field in pltpu.CompilerParams.__dataclass_fields__:
        _cp = dataclasses.replace(_cp, **{_field: _val})


@functools.partial(
    pl.kernel,
    out_type=jax.ShapeDtypeStruct((B * S,), jnp.float32),
    mesh=plsc.VectorSubcoreMesh(core_axis_name="c", subcore_axis_name="s"),
    compiler_params=_cp,
    scratch_types=[
        pltpu.VMEM((FLAT,), jnp.int32),
        pltpu.VMEM((FLAT,), jnp.int32),
        pltpu.VMEM((RPW, D), jnp.float32),
        pltpu.VMEM((SPAD, W2), jnp.float32),
        pltpu.VMEM((SPAD, W2), jnp.float32),
        pltpu.VMEM((SPAD,), jnp.float32),
        pltpu.VMEM((SPAD,), jnp.float32),
        pltpu.VMEM((OCHW + 8,), jnp.float32),
        pltpu.VMEM((OCHW + 8,), jnp.float32),
        pltpu.SemaphoreType.DMA,
        pltpu.SemaphoreType.DMA,
        pltpu.SemaphoreType.DMA,
        pltpu.SemaphoreType.DMA,
    ],
)
def _sc_sparse_linear(embed_hbm, gidx_hbm, pidx_hbm, weight_hbm, bias_hbm,
                      out_hbm, gidx_v, pidx_v, embed_v, rows0, rows1,
                      bv0, bv1, out0, out1, sem_in, sem_g0, sem_g1, sem_out):
    _sc_body(embed_hbm, gidx_hbm, pidx_hbm, weight_hbm, bias_hbm, out_hbm,
             gidx_v, pidx_v, embed_v, rows0, rows1, bv0, bv1,
             out0, out1, sem_in, sem_g0, sem_g1, sem_out)


@jax.jit
def kernel(embed, shortlist, weight, bias):
    b, s = shortlist.shape
    pidx = shortlist.astype(jnp.int32).reshape(-1)
    gidx = pidx >> 1
    weight2 = weight.reshape(weight.shape[0] // 2, 2 * weight.shape[1])
    bias_flat = bias.reshape(-1)
    out = _sc_sparse_linear(embed, gidx, pidx, weight2, bias_flat)
    return out.reshape(b, s)


# speedup vs baseline: 2.2078x; 1.0826x over previous
import dataclasses
import functools

import jax
import jax.numpy as jnp
from jax import lax
from jax.experimental import pallas as pl
from jax.experimental.pallas import tpu as pltpu
from jax.experimental.pallas import tpu_sc as plsc

B = 4096
S = 200
D = 64
L = 16
NC = 2
NS = 16
NW = NC * NS
RPW = B // NW
SPAD = ((S + L - 1) // L) * L
NG = SPAD // L
C1 = 128
C2 = S - C1
FLAT = RPW * S
W2 = 2 * D
OCH = 16
OCHW = OCH * S


def _sc_body(embed_hbm, pidx_hbm, weight_hbm, bias_hbm, out_hbm,
             pidx_v, embed_v, rows0, rows1, bv0, bv1,
             out0, out1, sem_in, sem_g0, sem_g1, sem_out):
    wid = lax.axis_index("s") * NC + lax.axis_index("c")
    row0 = wid * RPW
    base = wid * FLAT

    cp_p = pltpu.async_copy(pidx_hbm.at[pl.ds(base, FLAT)], pidx_v, sem_in)
    cp_e = pltpu.async_copy(embed_hbm.at[pl.ds(row0, RPW)], embed_v, sem_in)
    cp_p.wait()
    cp_e.wait()

    def fire(r, rows_v, bv, sem):
        off = r * S
        pltpu.async_copy(weight_hbm.at[pidx_v.at[pl.ds(off, C1)]],
                         rows_v.at[pl.ds(0, C1)], sem)
        pltpu.async_copy(weight_hbm.at[pidx_v.at[pl.ds(off + C1, C2)]],
                         rows_v.at[pl.ds(C1, C2)], sem)
        pltpu.async_copy(bias_hbm.at[pidx_v.at[pl.ds(off, C1)]],
                         bv.at[pl.ds(0, C1)], sem)
        pltpu.async_copy(bias_hbm.at[pidx_v.at[pl.ds(off + C1, C2)]],
                         bv.at[pl.ds(C1, C2)], sem)

    def drain(r, rows_v, bv, sem):
        off = r * S
        pltpu.make_async_copy(weight_hbm.at[pidx_v.at[pl.ds(off, C1)]],
                              rows_v.at[pl.ds(0, C1)], sem).wait()
        pltpu.make_async_copy(weight_hbm.at[pidx_v.at[pl.ds(off + C1, C2)]],
                              rows_v.at[pl.ds(C1, C2)], sem).wait()
        pltpu.make_async_copy(bias_hbm.at[pidx_v.at[pl.ds(off, C1)]],
                              bv.at[pl.ds(0, C1)], sem).wait()
        pltpu.make_async_copy(bias_hbm.at[pidx_v.at[pl.ds(off + C1, C2)]],
                              bv.at[pl.ds(C1, C2)], sem).wait()

    def lane_perm(x, idx):
        return lax.gather(
            x, idx[:, None],
            dimension_numbers=lax.GatherDimensionNumbers(
                offset_dims=(), collapsed_slice_dims=(0,),
                start_index_map=(0,)),
            slice_sizes=(1,),
            mode=lax.GatherScatterMode.PROMISE_IN_BOUNDS)

    bitrev = [int(f"{i:04b}"[::-1], 2) for i in range(L)]

    def compute(r, rows_v, bv, out_v):
        e = [embed_v[r, pl.ds(k * L, L)] for k in range(D // L)]

        @pl.loop(0, NG)
        def _(g):
            s_base = g * L
            accs = []
            for j in range(L):
                s = s_base + bitrev[j]
                w0 = rows_v[s, pl.ds(0, L)]
                acc = w0 * e[0]
                for k in range(1, D // L):
                    acc = acc + rows_v[s, pl.ds(k * L, L)] * e[k]
                accs.append(acc)
            lane = lax.iota(jnp.int32, L)
            for h2 in (8, 4, 2, 1):
                mask = (lane & h2) == 0
                xor_idx = lane ^ h2
                nxt = []
                for i in range(0, len(accs), 2):
                    a, b = accs[i], accs[i + 1]
                    nxt.append(jnp.where(mask,
                                         a + lane_perm(a, xor_idx),
                                         b + lane_perm(b, xor_idx)))
                accs = nxt
            loc = (r % OCH) * S + s_base
            out_v[pl.ds(loc, L)] = accs[0] + bv[pl.ds(s_base, L)]

    fire(0, rows0, bv0, sem_g0)

    @pl.loop(0, RPW // 2)
    def _(p):
        r0 = p * 2
        r1 = r0 + 1
        chunk_par = (r0 // OCH) % 2
        fire(r1, rows1, bv1, sem_g1)
        drain(r0, rows0, bv0, sem_g0)

        @pl.when(chunk_par == 0)
        def _():
            compute(r0, rows0, bv0, out0)

        @pl.when(chunk_par == 1)
        def _():
            compute(r0, rows0, bv0, out1)

        @pl.when(p < RPW // 2 - 1)
        def _():
            fire(r0 + 2, rows0, bv0, sem_g0)

        drain(r1, rows1, bv1, sem_g1)

        @pl.when(chunk_par == 0)
        def _():
            compute(r1, rows1, bv1, out0)

        @pl.when(chunk_par == 1)
        def _():
            compute(r1, rows1, bv1, out1)

        @pl.when(r1 % OCH == OCH - 1)
        def _():
            c = r1 // OCH

            @pl.when(c % 2 == 0)
            def _():
                pltpu.async_copy(out0.at[pl.ds(0, OCHW)],
                                 out_hbm.at[pl.ds(base + c * OCHW, OCHW)],
                                 sem_out)

            @pl.when(c % 2 == 1)
            def _():
                pltpu.async_copy(out1.at[pl.ds(0, OCHW)],
                                 out_hbm.at[pl.ds(base + c * OCHW, OCHW)],
                                 sem_out)

            @pl.when(c >= 1)
            def _():
                pltpu.make_async_copy(
                    out0.at[pl.ds(0, OCHW)],
                    out_hbm.at[pl.ds(base + (c - 1) * OCHW, OCHW)],
                    sem_out).wait()

    pltpu.make_async_copy(
        out1.at[pl.ds(0, OCHW)],
        out_hbm.at[pl.ds(base + (RPW // OCH - 1) * OCHW, OCHW)],
        sem_out).wait()


_cp = pltpu.CompilerParams()
for _field, _val in (("needs_layout_passes", False),
                     ("use_tc_tiling_on_sc", True)):
    if _field in pltpu.CompilerParams.__dataclass_fields__:
        _cp = dataclasses.replace(_cp, **{_field: _val})


@functools.partial(
    pl.kernel,
    out_type=jax.ShapeDtypeStruct((B * S,), jnp.float32),
    mesh=plsc.VectorSubcoreMesh(core_axis_name="c", subcore_axis_name="s"),
    compiler_params=_cp,
    scratch_types=[
        pltpu.VMEM((FLAT,), jnp.int32),
        pltpu.VMEM((RPW, D), jnp.float32),
        pltpu.VMEM((SPAD, W2), jnp.float32),
        pltpu.VMEM((SPAD, W2), jnp.float32),
        pltpu.VMEM((SPAD,), jnp.float32),
        pltpu.VMEM((SPAD,), jnp.float32),
        pltpu.VMEM((OCHW + 8,), jnp.float32),
        pltpu.VMEM((OCHW + 8,), jnp.float32),
        pltpu.SemaphoreType.DMA,
        pltpu.SemaphoreType.DMA,
        pltpu.SemaphoreType.DMA,
        pltpu.SemaphoreType.DMA,
    ],
)
def _sc_sparse_linear(embed_hbm, pidx_hbm, weight_hbm, bias_hbm,
                      out_hbm, pidx_v, embed_v, rows0, rows1,
                      bv0, bv1, out0, out1, sem_in, sem_g0, sem_g1, sem_out):
    _sc_body(embed_hbm, pidx_hbm, weight_hbm, bias_hbm, out_hbm,
             pidx_v, embed_v, rows0, rows1, bv0, bv1,
             out0, out1, sem_in, sem_g0, sem_g1, sem_out)


@jax.jit
def kernel(embed, shortlist, weight, bias):
    b, s = shortlist.shape
    pidx = shortlist.astype(jnp.int32).reshape(-1)
    weight2 = jnp.pad(weight, ((0, 0), (0, 2 * D - weight.shape[1])))
    bias_flat = bias.reshape(-1)
    out = _sc_sparse_linear(embed, pidx, weight2, bias_flat)
    return out.reshape(b, s)


# speedup vs baseline: 2.2080x; 1.0001x over previous
import dataclasses
import functools

import jax
import jax.numpy as jnp
from jax import lax
from jax.experimental import pallas as pl
from jax.experimental.pallas import tpu as pltpu
from jax.experimental.pallas import tpu_sc as plsc

B = 4096
S = 200
D = 64
L = 16
NC = 2
NS = 16
NW = NC * NS
RPW = B // NW
SPAD = ((S + L - 1) // L) * L
NG = SPAD // L
C1 = 128
C2 = S - C1
FLAT = RPW * S
W2 = 2 * D
OCH = 16
OCHW = OCH * S


def _sc_body(embed_hbm, pidx_hbm, weight_hbm, bias_hbm, out_hbm,
             pidx_v, embed_v, rows0, rows1, rows2, bv0, bv1, bv2,
             out0, out1, sem_in, sem_g0, sem_g1, sem_g2, sem_out):
    wid = lax.axis_index("s") * NC + lax.axis_index("c")
    row0 = wid * RPW
    base = wid * FLAT

    cp_p = pltpu.async_copy(pidx_hbm.at[pl.ds(base, FLAT)], pidx_v, sem_in)
    cp_e = pltpu.async_copy(embed_hbm.at[pl.ds(row0, RPW)], embed_v, sem_in)
    cp_p.wait()
    cp_e.wait()

    def fire(r, rows_v, bv, sem):
        off = r * S
        pltpu.async_copy(weight_hbm.at[pidx_v.at[pl.ds(off, C1)]],
                         rows_v.at[pl.ds(0, C1)], sem)
        pltpu.async_copy(weight_hbm.at[pidx_v.at[pl.ds(off + C1, C2)]],
                         rows_v.at[pl.ds(C1, C2)], sem)
        pltpu.async_copy(bias_hbm.at[pidx_v.at[pl.ds(off, C1)]],
                         bv.at[pl.ds(0, C1)], sem)
        pltpu.async_copy(bias_hbm.at[pidx_v.at[pl.ds(off + C1, C2)]],
                         bv.at[pl.ds(C1, C2)], sem)

    def drain(r, rows_v, bv, sem):
        off = r * S
        pltpu.make_async_copy(weight_hbm.at[pidx_v.at[pl.ds(off, C1)]],
                              rows_v.at[pl.ds(0, C1)], sem).wait()
        pltpu.make_async_copy(weight_hbm.at[pidx_v.at[pl.ds(off + C1, C2)]],
                              rows_v.at[pl.ds(C1, C2)], sem).wait()
        pltpu.make_async_copy(bias_hbm.at[pidx_v.at[pl.ds(off, C1)]],
                              bv.at[pl.ds(0, C1)], sem).wait()
        pltpu.make_async_copy(bias_hbm.at[pidx_v.at[pl.ds(off + C1, C2)]],
                              bv.at[pl.ds(C1, C2)], sem).wait()

    def lane_perm(x, idx):
        return lax.gather(
            x, idx[:, None],
            dimension_numbers=lax.GatherDimensionNumbers(
                offset_dims=(), collapsed_slice_dims=(0,),
                start_index_map=(0,)),
            slice_sizes=(1,),
            mode=lax.GatherScatterMode.PROMISE_IN_BOUNDS)

    bitrev = [int(f"{i:04b}"[::-1], 2) for i in range(L)]

    def compute(r, rows_v, bv, out_v):
        e = [embed_v[r, pl.ds(k * L, L)] for k in range(D // L)]

        @pl.loop(0, NG)
        def _(g):
            s_base = g * L
            accs = []
            for j in range(L):
                s = s_base + bitrev[j]
                w0 = rows_v[s, pl.ds(0, L)]
                acc = w0 * e[0]
                for k in range(1, D // L):
                    acc = acc + rows_v[s, pl.ds(k * L, L)] * e[k]
                accs.append(acc)
            lane = lax.iota(jnp.int32, L)
            for h2 in (8, 4, 2, 1):
                mask = (lane & h2) == 0
                xor_idx = lane ^ h2
                nxt = []
                for i in range(0, len(accs), 2):
                    a, b = accs[i], accs[i + 1]
                    nxt.append(jnp.where(mask,
                                         a + lane_perm(a, xor_idx),
                                         b + lane_perm(b, xor_idx)))
                accs = nxt
            loc = (r % OCH) * S + s_base
            out_v[pl.ds(loc, L)] = accs[0] + bv[pl.ds(s_base, L)]

    bufs = ((rows0, bv0, sem_g0), (rows1, bv1, sem_g1), (rows2, bv2, sem_g2))
    fire(0, *bufs[0])
    fire(1, *bufs[1])

    @pl.loop(0, RPW)
    def _(r):
        chunk_par = (r // OCH) % 2
        for q in range(3):
            @pl.when(r % 3 == q)
            def _():
                drain(r, *bufs[q])

                @pl.when(chunk_par == 0)
                def _():
                    compute(r, bufs[q][0], bufs[q][1], out0)

                @pl.when(chunk_par == 1)
                def _():
                    compute(r, bufs[q][0], bufs[q][1], out1)

                @pl.when(r + 2 < RPW)
                def _():
                    fire(r + 2, *bufs[(q + 2) % 3])

        @pl.when(r % OCH == OCH - 1)
        def _():
            c = r // OCH

            @pl.when(c % 2 == 0)
            def _():
                pltpu.async_copy(out0.at[pl.ds(0, OCHW)],
                                 out_hbm.at[pl.ds(base + c * OCHW, OCHW)],
                                 sem_out)

            @pl.when(c % 2 == 1)
            def _():
                pltpu.async_copy(out1.at[pl.ds(0, OCHW)],
                                 out_hbm.at[pl.ds(base + c * OCHW, OCHW)],
                                 sem_out)

            @pl.when(c >= 1)
            def _():
                pltpu.make_async_copy(
                    out0.at[pl.ds(0, OCHW)],
                    out_hbm.at[pl.ds(base + (c - 1) * OCHW, OCHW)],
                    sem_out).wait()

    pltpu.make_async_copy(
        out1.at[pl.ds(0, OCHW)],
        out_hbm.at[pl.ds(base + (RPW // OCH - 1) * OCHW, OCHW)],
        sem_out).wait()


_cp = pltpu.CompilerParams()
for _field, _val in (("needs_layout_passes", False),
                     ("use_tc_tiling_on_sc", True)):
    if _field in pltpu.CompilerParams.__dataclass_fields__:
        _cp = dataclasses.replace(_cp, **{_field: _val})


@functools.partial(
    pl.kernel,
    out_type=jax.ShapeDtypeStruct((B * S,), jnp.float32),
    mesh=plsc.VectorSubcoreMesh(core_axis_name="c", subcore_axis_name="s"),
    compiler_params=_cp,
    scratch_types=[
        pltpu.VMEM((FLAT,), jnp.int32),
        pltpu.VMEM((RPW, D), jnp.float32),
        pltpu.VMEM((SPAD, W2), jnp.float32),
        pltpu.VMEM((SPAD, W2), jnp.float32),
        pltpu.VMEM((SPAD, W2), jnp.float32),
        pltpu.VMEM((SPAD,), jnp.float32),
        pltpu.VMEM((SPAD,), jnp.float32),
        pltpu.VMEM((SPAD,), jnp.float32),
        pltpu.VMEM((OCHW + 8,), jnp.float32),
        pltpu.VMEM((OCHW + 8,), jnp.float32),
        pltpu.SemaphoreType.DMA,
        pltpu.SemaphoreType.DMA,
        pltpu.SemaphoreType.DMA,
        pltpu.SemaphoreType.DMA,
        pltpu.SemaphoreType.DMA,
    ],
)
def _sc_sparse_linear(embed_hbm, pidx_hbm, weight_hbm, bias_hbm,
                      out_hbm, pidx_v, embed_v, rows0, rows1, rows2,
                      bv0, bv1, bv2, out0, out1,
                      sem_in, sem_g0, sem_g1, sem_g2, sem_out):
    _sc_body(embed_hbm, pidx_hbm, weight_hbm, bias_hbm, out_hbm,
             pidx_v, embed_v, rows0, rows1, rows2, bv0, bv1, bv2,
             out0, out1, sem_in, sem_g0, sem_g1, sem_g2, sem_out)


@jax.jit
def kernel(embed, shortlist, weight, bias):
    b, s = shortlist.shape
    pidx = shortlist.astype(jnp.int32).reshape(-1)
    weight2 = jnp.pad(weight, ((0, 0), (0, 2 * D - weight.shape[1])))
    bias_flat = bias.reshape(-1)
    out = _sc_sparse_linear(embed, pidx, weight2, bias_flat)
    return out.reshape(b, s)


# speedup vs baseline: 2.3214x; 1.0514x over previous
import dataclasses
import functools

import jax
import jax.numpy as jnp
from jax import lax
from jax.experimental import pallas as pl
from jax.experimental.pallas import tpu as pltpu
from jax.experimental.pallas import tpu_sc as plsc

B = 4096
S = 200
D = 64
L = 16
NC = 2
NS = 16
NW = NC * NS
RPW = B // NW
SPAD = ((S + L - 1) // L) * L
NG = SPAD // L
C1 = 128
C2 = S - C1
FLAT = RPW * S


def _sc_body(embed_hbm, short_hbm, weight_hbm, bias_hbm, out_hbm,
             idx_v, embed_v, rows0, rows1, rows2, rows3,
             bv0, bv1, bv2, bv3, out_v,
             sem_in, sem_g0, sem_g1, sem_g2, sem_g3):
    wid = lax.axis_index("s") * NC + lax.axis_index("c")
    row0 = wid * RPW
    base = wid * FLAT

    cp_i = pltpu.async_copy(short_hbm.at[pl.ds(base, FLAT)], idx_v, sem_in)
    cp_e = pltpu.async_copy(embed_hbm.at[pl.ds(row0, RPW)], embed_v, sem_in)
    cp_i.wait()
    cp_e.wait()

    def fire(r, rows_v, bv, sem):
        off = r * S
        pltpu.async_copy(weight_hbm.at[idx_v.at[pl.ds(off, C1)]],
                         rows_v.at[pl.ds(0, C1)], sem)
        pltpu.async_copy(weight_hbm.at[idx_v.at[pl.ds(off + C1, C2)]],
                         rows_v.at[pl.ds(C1, C2)], sem)
        pltpu.async_copy(bias_hbm.at[idx_v.at[pl.ds(off, C1)]],
                         bv.at[pl.ds(0, C1)], sem)
        pltpu.async_copy(bias_hbm.at[idx_v.at[pl.ds(off + C1, C2)]],
                         bv.at[pl.ds(C1, C2)], sem)

    def drain(r, rows_v, bv, sem):
        off = r * S
        pltpu.make_async_copy(weight_hbm.at[idx_v.at[pl.ds(off, C1)]],
                              rows_v.at[pl.ds(0, C1)], sem).wait()
        pltpu.make_async_copy(weight_hbm.at[idx_v.at[pl.ds(off + C1, C2)]],
                              rows_v.at[pl.ds(C1, C2)], sem).wait()
        pltpu.make_async_copy(bias_hbm.at[idx_v.at[pl.ds(off, C1)]],
                              bv.at[pl.ds(0, C1)], sem).wait()
        pltpu.make_async_copy(bias_hbm.at[idx_v.at[pl.ds(off + C1, C2)]],
                              bv.at[pl.ds(C1, C2)], sem).wait()

    def lane_perm(x, idx):
        return lax.gather(
            x, idx[:, None],
            dimension_numbers=lax.GatherDimensionNumbers(
                offset_dims=(), collapsed_slice_dims=(0,),
                start_index_map=(0,)),
            slice_sizes=(1,),
            mode=lax.GatherScatterMode.PROMISE_IN_BOUNDS)

    bitrev = [int(f"{i:04b}"[::-1], 2) for i in range(L)]

    def compute(r, rows_v, bv):
        e = [embed_v[r, pl.ds(k * L, L)] for k in range(D // L)]

        @pl.loop(0, NG)
        def _(g):
            s_base = g * L
            accs = []
            for j in range(L):
                s = s_base + bitrev[j]
                w0 = rows_v[s, pl.ds(0, L)]
                acc = w0 * e[0]
                for k in range(1, D // L):
                    acc = acc + rows_v[s, pl.ds(k * L, L)] * e[k]
                accs.append(acc)
            lane = lax.iota(jnp.int32, L)
            for h in (8, 4, 2, 1):
                mask = (lane & h) == 0
                xor_idx = lane ^ h
                nxt = []
                for i in range(0, len(accs), 2):
                    a, b = accs[i], accs[i + 1]
                    nxt.append(jnp.where(mask,
                                         a + lane_perm(a, xor_idx),
                                         b + lane_perm(b, xor_idx)))
                accs = nxt
            out_v[pl.ds(r * S + s_base, L)] = accs[0] + bv[pl.ds(s_base, L)]

    bufs = ((rows0, bv0, sem_g0), (rows1, bv1, sem_g1),
            (rows2, bv2, sem_g2), (rows3, bv3, sem_g3))
    for q in range(3):
        fire(q, *bufs[q])

    @pl.loop(0, RPW // 4)
    def _(p):
        for q in range(4):
            r = p * 4 + q
            drain(r, *bufs[q])
            compute(r, *bufs[q][:2])

            @pl.when(r + 3 < RPW)
            def _():
                fire(r + 3, *bufs[(q + 3) % 4])

    pltpu.sync_copy(out_v.at[pl.ds(0, FLAT)], out_hbm.at[pl.ds(base, FLAT)])


_cp = pltpu.CompilerParams()
for _field, _val in (("needs_layout_passes", False),
                     ("use_tc_tiling_on_sc", False)):
    if _field in pltpu.CompilerParams.__dataclass_fields__:
        _cp = dataclasses.replace(_cp, **{_field: _val})


@functools.partial(
    pl.kernel,
    out_type=jax.ShapeDtypeStruct((B * S,), jnp.float32),
    mesh=plsc.VectorSubcoreMesh(core_axis_name="c", subcore_axis_name="s"),
    compiler_params=_cp,
    scratch_types=[
        pltpu.VMEM((FLAT,), jnp.int32),
        pltpu.VMEM((RPW, D), jnp.float32),
        pltpu.VMEM((SPAD, D), jnp.float32),
        pltpu.VMEM((SPAD, D), jnp.float32),
        pltpu.VMEM((SPAD, D), jnp.float32),
        pltpu.VMEM((SPAD, D), jnp.float32),
        pltpu.VMEM((SPAD,), jnp.float32),
        pltpu.VMEM((SPAD,), jnp.float32),
        pltpu.VMEM((SPAD,), jnp.float32),
        pltpu.VMEM((SPAD,), jnp.float32),
        pltpu.VMEM((FLAT + 8,), jnp.float32),
        pltpu.SemaphoreType.DMA,
        pltpu.SemaphoreType.DMA,
        pltpu.SemaphoreType.DMA,
        pltpu.SemaphoreType.DMA,
        pltpu.SemaphoreType.DMA,
    ],
)
def _sc_sparse_linear(embed_hbm, short_hbm, weight_hbm, bias_hbm, out_hbm,
                      idx_v, embed_v, rows0, rows1, rows2, rows3,
                      bv0, bv1, bv2, bv3, out_v,
                      sem_in, sem_g0, sem_g1, sem_g2, sem_g3):
    _sc_body(embed_hbm, short_hbm, weight_hbm, bias_hbm, out_hbm,
             idx_v, embed_v, rows0, rows1, rows2, rows3,
             bv0, bv1, bv2, bv3, out_v,
             sem_in, sem_g0, sem_g1, sem_g2, sem_g3)


@jax.jit
def kernel(embed, shortlist, weight, bias):
    b, s = shortlist.shape
    short_flat = shortlist.astype(jnp.int32).reshape(-1)
    bias_flat = bias.reshape(-1)
    out = _sc_sparse_linear(embed, short_flat, weight, bias_flat)
    return out.reshape(b, s)
